# baseline (speedup 1.0000x reference)
import functools

import jax
import jax.numpy as jnp
from jax import lax
from jax.experimental import pallas as pl
from jax.experimental.pallas import tpu as pltpu
from jax.experimental.pallas import tpu_sc as plsc

B = 16384
F = 26
V = 100000
D = 16
NUM = 16

NC = 2
NS = 16
NW = NC * NS
R = F * D
R1 = 256
QC = 4096
NQ = B // QC
UNROLL = 8


def _make_gather_body(row0, npairs):
    def _gather_body(tbl_hbm, xcat_hbm, out_hbm, row_v, idx_v, res0, res1,
                     rsem, isem, os0, os1):
        wid = lax.axis_index("s") * NC + lax.axis_index("c")
        @pl.when((lax.axis_index("s") & 1) == 1)
        def _stagger():
            pl.delay(3400)

        res_b = (res0, res1)
        osem = (os0, os1)

        def pair_body(j, _):
            p = row0 + wid * npairs + j
            f = p >> 4
            row_cp = pltpu.async_copy(tbl_hbm.at[p], row_v, rsem)

            @pl.when((j == 0) | ((p & 15) == 0))
            def _load_idx():
                pltpu.async_copy(xcat_hbm.at[f], idx_v, isem).wait()

            row_cp.wait()
            ocp = [None, None]
            for t in range(NQ):
                b = t & 1
                if ocp[b] is not None:
                    ocp[b].wait()
                rv = res_b[b]

                def gather_step(i, _):
                    base = t * QC + i * (16 * UNROLL)
                    for k in range(UNROLL):
                        o = base + k * 16
                        vec = idx_v[pl.ds(o, 16)]
                        rv[pl.ds(o - t * QC, 16)] = plsc.load_gather(
                            row_v, [vec])
                    return 0

                lax.fori_loop(0, QC // (16 * UNROLL), gather_step, 0)
                ocp[b] = pltpu.async_copy(
                    rv, out_hbm.at[p - row0, pl.ds(t * QC, QC)], osem[b])
            ocp[0].wait()
            ocp[1].wait()
            return 0

        lax.fori_loop(0, npairs, pair_body, 0)

    return _gather_body


def _sc_gather(tblT, xcatT, row0, nrows):
    mesh = plsc.VectorSubcoreMesh(core_axis_name="c", subcore_axis_name="s")
    f = pl.kernel(
        _make_gather_body(row0, nrows // NW),
        out_type=jax.ShapeDtypeStruct((nrows, B), jnp.float32),
        mesh=mesh,
        compiler_params=pltpu.CompilerParams(use_tc_tiling_on_sc=True,
                                             needs_layout_passes=False),
        scratch_types=[
            pltpu.VMEM((V,), jnp.float32),
            pltpu.VMEM((B,), jnp.int32),
            pltpu.VMEM((QC,), jnp.float32),
            pltpu.VMEM((QC,), jnp.float32),
            pltpu.SemaphoreType.DMA,
            pltpu.SemaphoreType.DMA,
            pltpu.SemaphoreType.DMA,
            pltpu.SemaphoreType.DMA,
        ],
    )
    return f(tblT, xcatT)


def _mlp1_body(xnt_ref, e1_ref, w1t_ref, b1_ref, h_ref):
    h = jnp.dot(w1t_ref[:, :NUM], xnt_ref[...],
                preferred_element_type=jnp.float32)
    h = h + jnp.dot(w1t_ref[:, NUM:NUM + R1], e1_ref[...],
                    preferred_element_type=jnp.float32)
    h_ref[...] = h + b1_ref[...]


def _mlp2_body(h_ref, e2_ref, w1t_ref, w2t_ref, b2_ref, w3t_ref, b3_ref,
               o_ref):
    h = h_ref[...] + jnp.dot(w1t_ref[:, NUM + R1:], e2_ref[...],
                             preferred_element_type=jnp.float32)
    h = jnp.maximum(h, 0.0)
    h = jnp.dot(w2t_ref[...], h, preferred_element_type=jnp.float32)
    h = jnp.maximum(h + b2_ref[...], 0.0)
    o_ref[...] = jnp.dot(w3t_ref[...], h,
                         preferred_element_type=jnp.float32) + b3_ref[...]


def _tc_mlp1(xnT, embsT1, W1T, b1):
    BK = 4096
    in_dim = NUM + R
    return pl.pallas_call(
        _mlp1_body,
        grid=(B // BK,),
        in_specs=[
            pl.BlockSpec((NUM, BK), lambda i: (0, i)),
            pl.BlockSpec((R1, BK), lambda i: (0, i)),
            pl.BlockSpec((64, in_dim), lambda i: (0, 0)),
            pl.BlockSpec((64, 1), lambda i: (0, 0)),
        ],
        out_specs=pl.BlockSpec((64, BK), lambda i: (0, i)),
        out_shape=jax.ShapeDtypeStruct((64, B), jnp.float32),
    )(xnT, embsT1, W1T, b1.reshape(64, 1))


def _tc_mlp2(hpre, embsT2, W1T, W2T, b2, W3T, b3):
    BK = 4096
    in_dim = NUM + R
    return pl.pallas_call(
        _mlp2_body,
        grid=(B // BK,),
        in_specs=[
            pl.BlockSpec((64, BK), lambda i: (0, i)),
            pl.BlockSpec((R - R1, BK), lambda i: (0, i)),
            pl.BlockSpec((64, in_dim), lambda i: (0, 0)),
            pl.BlockSpec((32, 64), lambda i: (0, 0)),
            pl.BlockSpec((32, 1), lambda i: (0, 0)),
            pl.BlockSpec((1, 32), lambda i: (0, 0)),
            pl.BlockSpec((1, 1), lambda i: (0, 0)),
        ],
        out_specs=pl.BlockSpec((1, BK), lambda i: (0, i)),
        out_shape=jax.ShapeDtypeStruct((1, B), jnp.float32),
    )(hpre, embsT2, W1T, W2T, b2.reshape(32, 1), W3T, b3.reshape(1, 1))


def kernel(x_num, x_cat, emb_tables, W1, b1, W2, b2, W3, b3):
    tblT = emb_tables.transpose(0, 2, 1).reshape(R, V)
    xcatT = x_cat.T.astype(jnp.int32)
    embsT1 = _sc_gather(tblT, xcatT, 0, R1)
    embsT2 = _sc_gather(tblT, xcatT, R1, R - R1)
    hpre = _tc_mlp1(x_num.T, embsT1, W1.T, b1)
    outT = _tc_mlp2(hpre, embsT2, W1.T, W2.T, b2, W3.T, b3)
    return outT.reshape(B, 1)


# speedup vs baseline: 1.0389x; 1.0389x over previous
import functools

import jax
import jax.numpy as jnp
from jax import lax
from jax.experimental import pallas as pl
from jax.experimental.pallas import tpu as pltpu
from jax.experimental.pallas import tpu_sc as plsc

B = 16384
F = 26
V = 100000
D = 16
NUM = 16

NC = 2
NS = 16
NW = NC * NS
R = F * D
PW = R // NW
QC = 4096
NQ = B // QC
UNROLL = 16


def _gather_body(tbl_hbm, xcat_hbm, out_hbm, row_v, idx_v, res0, res1,
                 rsem, isem, os0, os1):
    wid = lax.axis_index("s") * NC + lax.axis_index("c")
    @pl.when((lax.axis_index("s") & 1) == 1)
    def _stagger():
        pl.delay(3400)

    res_b = (res0, res1)
    osem = (os0, os1)

    def pair_body(j, _):
        p = wid * PW + j
        f = p >> 4
        row_cp = pltpu.async_copy(tbl_hbm.at[p], row_v, rsem)

        @pl.when((j == 0) | ((p & 15) == 0))
        def _load_idx():
            pltpu.async_copy(xcat_hbm.at[f], idx_v, isem).wait()

        row_cp.wait()
        ocp = [None, None]
        for t in range(NQ):
            b = t & 1
            if ocp[b] is not None:
                ocp[b].wait()
            rv = res_b[b]

            def gather_step(i, _):
                base = t * QC + i * (16 * UNROLL)
                for k in range(UNROLL):
                    o = base + k * 16
                    vec = idx_v[pl.ds(o, 16)]
                    rv[pl.ds(o - t * QC, 16)] = plsc.load_gather(row_v, [vec])
                return 0

            lax.fori_loop(0, QC // (16 * UNROLL), gather_step, 0)
            ocp[b] = pltpu.async_copy(
                rv, out_hbm.at[p, pl.ds(t * QC, QC)], osem[b])
        ocp[0].wait()
        ocp[1].wait()
        return 0

    lax.fori_loop(0, PW, pair_body, 0)


def _sc_gather(tblT, xcatT):
    mesh = plsc.VectorSubcoreMesh(core_axis_name="c", subcore_axis_name="s")
    f = pl.kernel(
        _gather_body,
        out_type=jax.ShapeDtypeStruct((R, B), jnp.float32),
        mesh=mesh,
        compiler_params=pltpu.CompilerParams(use_tc_tiling_on_sc=True,
                                             needs_layout_passes=False),
        scratch_types=[
            pltpu.VMEM((V,), jnp.float32),
            pltpu.VMEM((B,), jnp.int32),
            pltpu.VMEM((QC,), jnp.float32),
            pltpu.VMEM((QC,), jnp.float32),
            pltpu.SemaphoreType.DMA,
            pltpu.SemaphoreType.DMA,
            pltpu.SemaphoreType.DMA,
            pltpu.SemaphoreType.DMA,
        ],
    )
    return f(tblT, xcatT)


def _mlp_body(xnt_ref, et_ref, w1t_ref, b1_ref, w2t_ref, b2_ref, w3t_ref,
              b3_ref, o_ref):
    hp = jax.lax.Precision.DEFAULT
    h = jnp.dot(w1t_ref[:, :NUM], xnt_ref[...],
                preferred_element_type=jnp.float32, precision=hp)
    h = h + jnp.dot(w1t_ref[:, NUM:], et_ref[...],
                    preferred_element_type=jnp.float32, precision=hp)
    h = jnp.maximum(h + b1_ref[...], 0.0)
    h = jnp.dot(w2t_ref[...], h, preferred_element_type=jnp.float32,
                precision=hp)
    h = jnp.maximum(h + b2_ref[...], 0.0)
    o_ref[...] = jnp.dot(w3t_ref[...], h, preferred_element_type=jnp.float32,
                         precision=hp) + b3_ref[...]


def _tc_mlp(xnT, embsT, W1T, b1, W2T, b2, W3T, b3):
    BK = 4096
    in_dim = NUM + R
    grid = (B // BK,)
    return pl.pallas_call(
        _mlp_body,
        grid=grid,
        in_specs=[
            pl.BlockSpec((NUM, BK), lambda i: (0, i)),
            pl.BlockSpec((R, BK), lambda i: (0, i)),
            pl.BlockSpec((64, in_dim), lambda i: (0, 0)),
            pl.BlockSpec((64, 1), lambda i: (0, 0)),
            pl.BlockSpec((32, 64), lambda i: (0, 0)),
            pl.BlockSpec((32, 1), lambda i: (0, 0)),
            pl.BlockSpec((1, 32), lambda i: (0, 0)),
            pl.BlockSpec((1, 1), lambda i: (0, 0)),
        ],
        out_specs=pl.BlockSpec((1, BK), lambda i: (0, i)),
        out_shape=jax.ShapeDtypeStruct((1, B), jnp.float32),
    )(xnT, embsT, W1T, b1.reshape(64, 1), W2T, b2.reshape(32, 1), W3T,
      b3.reshape(1, 1))


def kernel(x_num, x_cat, emb_tables, W1, b1, W2, b2, W3, b3):
    tblT = emb_tables.transpose(0, 2, 1).reshape(R, V)
    xcatT = x_cat.T.astype(jnp.int32)
    embsT = _sc_gather(tblT, xcatT)
    outT = _tc_mlp(x_num.T, embsT, W1.T, b1, W2.T, b2, W3.T, b3)
    return outT.reshape(B, 1)
